# Initial kernel scaffold; baseline (speedup 1.0000x reference)
import jax
import jax.numpy as jnp
from jax.experimental import pallas as pl


def kernel(X, hash_table):
    raise NotImplementedError("write your pallas kernel here")


# speedup vs baseline: 21.8098x; 21.8098x over previous
import numpy as np
import jax
import jax.numpy as jnp
from jax import lax
from jax.experimental import pallas as pl
from jax.experimental.pallas import tpu as pltpu
from jax.experimental.pallas import tpu_sc as plsc

N = 1048576
T = 4194304
F = 2
_RES_MIN, _RES_MAX, _L, _LAYER = 16.0, 512.0, 16, 8
_GROWTH = np.exp((np.log(_RES_MAX) - np.log(_RES_MIN)) / (_L - 1))
RES = float(_GROWTH ** _LAYER * _RES_MIN)
SCALE = np.float32(RES - 1.0)
P1 = np.int32(2654435761 - (1 << 32))
P2 = np.int32(805459861)
MASK = np.int32(T - 1)

NC = 2
NS = 16
NW = NC * NS
NP = N // NW
B = 128
SB = 4096
CPS = SB // B
NSUPER = NP // SB


def _sc_body(xt, table8, out, x_buf, w_buf, row_bufs, hash_bufs, val_bufs,
             out_buf, sem):
    wid = lax.axis_index("s") * NC + lax.axis_index("c")
    lane = lax.iota(jnp.int32, 16)
    half = lane >> 1
    fbit = lane & 1
    one = jnp.float32(1.0)

    def super_body(s, carry):
        sbase = wid * NP + s * SB
        for d in range(3):
            pltpu.sync_copy(xt.at[pl.ds(d * N + sbase, SB)],
                            x_buf.at[pl.ds(d * SB, SB)])

        def chunk_body(k, carry2):
            cbase = k * B
            for g in range(B // 16):
                off = cbase + g * 16
                woff = g * 16
                s0 = x_buf[pl.ds(off, 16)] * SCALE
                s1 = x_buf[pl.ds(SB + off, 16)] * SCALE
                s2 = x_buf[pl.ds(2 * SB + off, 16)] * SCALE
                f0 = s0.astype(jnp.int32)
                f1 = s1.astype(jnp.int32)
                f2 = s2.astype(jnp.int32)
                f0 = jnp.where(f0.astype(jnp.float32) > s0, f0 - 1, f0)
                f1 = jnp.where(f1.astype(jnp.float32) > s1, f1 - 1, f1)
                f2 = jnp.where(f2.astype(jnp.float32) > s2, f2 - 1, f2)
                w_buf[pl.ds(woff, 16)] = s0 - f0.astype(jnp.float32)
                w_buf[pl.ds(B + woff, 16)] = s1 - f1.astype(jnp.float32)
                w_buf[pl.ds(2 * B + woff, 16)] = s2 - f2.astype(jnp.float32)
                h0 = (f0, f0 + 1)
                h1f = f1 * P1
                h1 = (h1f, h1f + P1)
                h2f = f2 * P2
                h2 = (h2f, h2f + P2)
                for c in range(8):
                    h = (h0[(c >> 2) & 1] ^ h1[(c >> 1) & 1] ^ h2[c & 1]) & MASK
                    hash_bufs[c][pl.ds(woff, 16)] = h
                    row_bufs[c][pl.ds(woff, 16)] = h >> 2
            copies = [
                pltpu.async_copy(table8.at[row_bufs[c]], val_bufs[c], sem)
                for c in range(8)
            ]
            for cp in copies:
                cp.wait()
            for j in range(B // 8):
                rows = half + j * 8
                w0 = plsc.load_gather(w_buf, [rows])
                w1 = plsc.load_gather(w_buf, [rows + B])
                w2 = plsc.load_gather(w_buf, [rows + 2 * B])
                v = []
                for c in range(8):
                    hd = plsc.load_gather(hash_bufs[c], [rows])
                    cols = ((hd & 3) << 1) + fbit
                    v.append(plsc.load_gather(val_bufs[c], [rows, cols]))
                p00 = v[0] * (one - w0) + v[4] * w0
                p01 = v[1] * (one - w0) + v[5] * w0
                p10 = v[2] * (one - w0) + v[6] * w0
                p11 = v[3] * (one - w0) + v[7] * w0
                p0 = p00 * (one - w1) + p10 * w1
                p1 = p01 * (one - w1) + p11 * w1
                res = p0 * (one - w2) + p1 * w2
                out_buf[pl.ds(cbase * 2 + j * 16, 16)] = res
            return carry2

        lax.fori_loop(jnp.int32(0), jnp.int32(CPS), chunk_body, jnp.int32(0))
        pltpu.sync_copy(out_buf, out.at[pl.ds(2 * sbase, 2 * SB)])
        return carry

    lax.fori_loop(jnp.int32(0), jnp.int32(NSUPER), super_body, jnp.int32(0))


@jax.jit
def _run(xt_flat, table8):
    fn = pl.kernel(
        _sc_body,
        out_type=jax.ShapeDtypeStruct((2 * N,), jnp.float32),
        mesh=plsc.VectorSubcoreMesh(
            core_axis_name="c", subcore_axis_name="s",
            num_cores=NC, num_subcores=NS,
        ),
        scratch_types=[
            pltpu.VMEM((3 * SB,), jnp.float32),
            pltpu.VMEM((3 * B,), jnp.float32),
            [pltpu.VMEM((B,), jnp.int32) for _ in range(8)],
            [pltpu.VMEM((B,), jnp.int32) for _ in range(8)],
            [pltpu.VMEM((B, 8), jnp.float32) for _ in range(8)],
            pltpu.VMEM((2 * SB,), jnp.float32),
            pltpu.SemaphoreType.DMA,
        ],
        compiler_params=pltpu.CompilerParams(
            needs_layout_passes=False, use_tc_tiling_on_sc=False
        ),
    )
    return fn(xt_flat, table8)


def kernel(X, hash_table):
    with jax.enable_x64(False):
        xt_flat = X.T.reshape(3 * N).astype(jnp.float32)
        table8 = hash_table.reshape(T // 4, 8)
        out_flat = _run(xt_flat, table8)
        return out_flat.reshape(N, F)


# speedup vs baseline: 22.2605x; 1.0207x over previous
import numpy as np
import jax
import jax.numpy as jnp
from jax import lax
from jax.experimental import pallas as pl
from jax.experimental.pallas import tpu as pltpu
from jax.experimental.pallas import tpu_sc as plsc

N = 1048576
T = 4194304
F = 2
_RES_MIN, _RES_MAX, _L, _LAYER = 16.0, 512.0, 16, 8
_GROWTH = np.exp((np.log(_RES_MAX) - np.log(_RES_MIN)) / (_L - 1))
RES = float(_GROWTH ** _LAYER * _RES_MIN)
SCALE = np.float32(RES - 1.0)
P1 = np.int32(2654435761 - (1 << 32))
P2 = np.int32(805459861)
MASK = np.int32(T - 1)

NC = 2
NS = 16
NW = NC * NS
NP = N // NW
B = 128
SB = 4096
CPS = SB // B
NSUPER = NP // SB


def _sc_body(xt, table8, out, x_buf, w_bufs, row_bufs, hash_bufs, val_bufs,
             out_buf, sem):
    wid = lax.axis_index("s") * NC + lax.axis_index("c")
    lane = lax.iota(jnp.int32, 16)
    half = lane >> 1
    fbit = lane & 1
    one = jnp.float32(1.0)

    def hash_issue(k, p):
        cbase = k * B
        for g in range(B // 16):
            off = cbase + g * 16
            woff = g * 16
            s0 = x_buf[pl.ds(off, 16)] * SCALE
            s1 = x_buf[pl.ds(SB + off, 16)] * SCALE
            s2 = x_buf[pl.ds(2 * SB + off, 16)] * SCALE
            f0 = s0.astype(jnp.int32)
            f1 = s1.astype(jnp.int32)
            f2 = s2.astype(jnp.int32)
            f0 = jnp.where(f0.astype(jnp.float32) > s0, f0 - 1, f0)
            f1 = jnp.where(f1.astype(jnp.float32) > s1, f1 - 1, f1)
            f2 = jnp.where(f2.astype(jnp.float32) > s2, f2 - 1, f2)
            w_bufs[p][pl.ds(woff, 16)] = s0 - f0.astype(jnp.float32)
            w_bufs[p][pl.ds(B + woff, 16)] = s1 - f1.astype(jnp.float32)
            w_bufs[p][pl.ds(2 * B + woff, 16)] = s2 - f2.astype(jnp.float32)
            h0 = (f0, f0 + 1)
            h1f = f1 * P1
            h1 = (h1f, h1f + P1)
            h2f = f2 * P2
            h2 = (h2f, h2f + P2)
            for c in range(8):
                h = (h0[(c >> 2) & 1] ^ h1[(c >> 1) & 1] ^ h2[c & 1]) & MASK
                hash_bufs[p][pl.ds(c * B + woff, 16)] = h
                row_bufs[p][pl.ds(c * B + woff, 16)] = h >> 2
        pltpu.async_copy(table8.at[row_bufs[p]], val_bufs[p], sem[p])

    def wait_gather(p):
        pltpu.make_async_copy(table8.at[row_bufs[p]], val_bufs[p],
                              sem[p]).wait()

    def interp(k, p):
        cbase = k * B
        for j in range(B // 8):
            rows = half + j * 8
            w0 = plsc.load_gather(w_bufs[p], [rows])
            w1 = plsc.load_gather(w_bufs[p], [rows + B])
            w2 = plsc.load_gather(w_bufs[p], [rows + 2 * B])
            v = []
            for c in range(8):
                hd = plsc.load_gather(hash_bufs[p], [rows + c * B])
                cols = ((hd & 3) << 1) + fbit
                v.append(plsc.load_gather(val_bufs[p], [rows + c * B, cols]))
            p00 = v[0] * (one - w0) + v[4] * w0
            p01 = v[1] * (one - w0) + v[5] * w0
            p10 = v[2] * (one - w0) + v[6] * w0
            p11 = v[3] * (one - w0) + v[7] * w0
            p0 = p00 * (one - w1) + p10 * w1
            p1 = p01 * (one - w1) + p11 * w1
            res = p0 * (one - w2) + p1 * w2
            out_buf[pl.ds(cbase * 2 + j * 16, 16)] = res

    def super_body(s, carry):
        sbase = wid * NP + s * SB
        for d in range(3):
            pltpu.sync_copy(xt.at[pl.ds(d * N + sbase, SB)],
                            x_buf.at[pl.ds(d * SB, SB)])

        hash_issue(jnp.int32(0), 0)

        def pair_body(m, carry2):
            k0 = m * 2
            hash_issue(k0 + 1, 1)
            wait_gather(0)
            interp(k0, 0)

            @pl.when(m < CPS // 2 - 1)
            def _():
                hash_issue(k0 + 2, 0)

            wait_gather(1)
            interp(k0 + 1, 1)
            return carry2

        lax.fori_loop(jnp.int32(0), jnp.int32(CPS // 2), pair_body,
                      jnp.int32(0))
        pltpu.sync_copy(out_buf, out.at[pl.ds(2 * sbase, 2 * SB)])
        return carry

    lax.fori_loop(jnp.int32(0), jnp.int32(NSUPER), super_body, jnp.int32(0))


@jax.jit
def _run(xt_flat, table8):
    fn = pl.kernel(
        _sc_body,
        out_type=jax.ShapeDtypeStruct((2 * N,), jnp.float32),
        mesh=plsc.VectorSubcoreMesh(
            core_axis_name="c", subcore_axis_name="s",
            num_cores=NC, num_subcores=NS,
        ),
        scratch_types=[
            pltpu.VMEM((3 * SB,), jnp.float32),
            [pltpu.VMEM((3 * B,), jnp.float32) for _ in range(2)],
            [pltpu.VMEM((8 * B,), jnp.int32) for _ in range(2)],
            [pltpu.VMEM((8 * B,), jnp.int32) for _ in range(2)],
            [pltpu.VMEM((8 * B, 8), jnp.float32) for _ in range(2)],
            pltpu.VMEM((2 * SB,), jnp.float32),
            [pltpu.SemaphoreType.DMA for _ in range(2)],
        ],
        compiler_params=pltpu.CompilerParams(
            needs_layout_passes=False, use_tc_tiling_on_sc=False
        ),
    )
    return fn(xt_flat, table8)


def kernel(X, hash_table):
    with jax.enable_x64(False):
        xt_flat = X.T.reshape(3 * N).astype(jnp.float32)
        table8 = hash_table.reshape(T // 4, 8)
        out_flat = _run(xt_flat, table8)
        return out_flat.reshape(N, F)


# speedup vs baseline: 101.5609x; 4.5624x over previous
import numpy as np
import jax
import jax.numpy as jnp
from jax import lax
from jax.experimental import pallas as pl
from jax.experimental.pallas import tpu as pltpu
from jax.experimental.pallas import tpu_sc as plsc

N = 1048576
T = 4194304
F = 2
_RES_MIN, _RES_MAX, _L, _LAYER = 16.0, 512.0, 16, 8
_GROWTH = np.exp((np.log(_RES_MAX) - np.log(_RES_MIN)) / (_L - 1))
RES = float(_GROWTH ** _LAYER * _RES_MIN)
SCALE = np.float32(RES - 1.0)
P1 = np.int32(2654435761 - (1 << 32))
P2 = np.int32(805459861)
MASK = np.int32(T - 1)

NC = 2
NS = 16
NW = NC * NS
NP = N // NW
B = 128
SB = 4096
CPS = SB // B
NSUPER = NP // SB


def _sc_body(xt, tq, out, x_buf, w_bufs, adr_bufs, val_bufs, out_buf, sem):
    wid = lax.axis_index("s") * NC + lax.axis_index("c")
    lane = lax.iota(jnp.int32, 16)
    one = jnp.float32(1.0)

    def hash_issue(k, p):
        cbase = k * B
        for g in range(B // 16):
            off = cbase + g * 16
            woff = g * 16
            s0 = x_buf[pl.ds(off, 16)] * SCALE
            s1 = x_buf[pl.ds(SB + off, 16)] * SCALE
            s2 = x_buf[pl.ds(2 * SB + off, 16)] * SCALE
            f0 = s0.astype(jnp.int32)
            f1 = s1.astype(jnp.int32)
            f2 = s2.astype(jnp.int32)
            f0 = jnp.where(f0.astype(jnp.float32) > s0, f0 - 1, f0)
            f1 = jnp.where(f1.astype(jnp.float32) > s1, f1 - 1, f1)
            f2 = jnp.where(f2.astype(jnp.float32) > s2, f2 - 1, f2)
            w_bufs[p][pl.ds(woff, 16)] = s0 - f0.astype(jnp.float32)
            w_bufs[p][pl.ds(B + woff, 16)] = s1 - f1.astype(jnp.float32)
            w_bufs[p][pl.ds(2 * B + woff, 16)] = s2 - f2.astype(jnp.float32)
            h0 = (f0, f0 + 1)
            h1f = f1 * P1
            h1 = (h1f, h1f + P1)
            h2f = f2 * P2
            h2 = (h2f, h2f + P2)
            for c in range(8):
                h = (h0[(c >> 2) & 1] ^ h1[(c >> 1) & 1] ^ h2[c & 1]) & MASK
                a0 = ((h >> 7) << 8) + (h & 127)
                adr_bufs[p][pl.ds(c * B + woff, 16)] = a0
                adr_bufs[p][pl.ds(8 * B + c * B + woff, 16)] = a0 + 128
        pltpu.async_copy(tq.at[adr_bufs[p]], val_bufs[p], sem[p])

    def wait_gather(p):
        pltpu.make_async_copy(tq.at[adr_bufs[p]], val_bufs[p], sem[p]).wait()

    def interp(k, p):
        cbase = k * B
        for j in range(B // 16):
            jo = j * 16
            w0 = w_bufs[p][pl.ds(jo, 16)]
            w1 = w_bufs[p][pl.ds(B + jo, 16)]
            w2 = w_bufs[p][pl.ds(2 * B + jo, 16)]
            u0, u1, u2 = one - w0, one - w1, one - w2
            res = []
            for f in range(2):
                v = [val_bufs[p][pl.ds(f * 8 * B + c * B + jo, 16)]
                     for c in range(8)]
                p00 = v[0] * u0 + v[4] * w0
                p01 = v[1] * u0 + v[5] * w0
                p10 = v[2] * u0 + v[6] * w0
                p11 = v[3] * u0 + v[7] * w0
                p0 = p00 * u1 + p10 * w1
                p1 = p01 * u1 + p11 * w1
                res.append(p0 * u2 + p1 * w2)
            pos = (cbase + jo) * 2 + lane * 2
            plsc.store_scatter(out_buf, [pos], res[0])
            plsc.store_scatter(out_buf, [pos + 1], res[1])

    def super_body(s, carry):
        sbase = wid * NP + s * SB
        for d in range(3):
            pltpu.sync_copy(xt.at[pl.ds(d * N + sbase, SB)],
                            x_buf.at[pl.ds(d * SB, SB)])

        hash_issue(jnp.int32(0), 0)

        def pair_body(m, carry2):
            k0 = m * 2
            hash_issue(k0 + 1, 1)
            wait_gather(0)
            interp(k0, 0)

            @pl.when(m < CPS // 2 - 1)
            def _():
                hash_issue(k0 + 2, 0)

            wait_gather(1)
            interp(k0 + 1, 1)
            return carry2

        lax.fori_loop(jnp.int32(0), jnp.int32(CPS // 2), pair_body,
                      jnp.int32(0))
        pltpu.sync_copy(out_buf, out.at[pl.ds(2 * sbase, 2 * SB)])
        return carry

    lax.fori_loop(jnp.int32(0), jnp.int32(NSUPER), super_body, jnp.int32(0))


@jax.jit
def _run(xt_flat, table8):
    fn = pl.kernel(
        _sc_body,
        out_type=jax.ShapeDtypeStruct((2 * N,), jnp.float32),
        mesh=plsc.VectorSubcoreMesh(
            core_axis_name="c", subcore_axis_name="s",
            num_cores=NC, num_subcores=NS,
        ),
        scratch_types=[
            pltpu.VMEM((3 * SB,), jnp.float32),
            [pltpu.VMEM((3 * B,), jnp.float32) for _ in range(2)],
            [pltpu.VMEM((16 * B,), jnp.int32) for _ in range(2)],
            [pltpu.VMEM((16 * B,), jnp.float32) for _ in range(2)],
            pltpu.VMEM((2 * SB,), jnp.float32),
            [pltpu.SemaphoreType.DMA for _ in range(2)],
        ],
        compiler_params=pltpu.CompilerParams(
            needs_layout_passes=False, use_tc_tiling_on_sc=False
        ),
    )
    return fn(xt_flat, table8)


def kernel(X, hash_table):
    with jax.enable_x64(False):
        xt_flat = X.T.reshape(3 * N).astype(jnp.float32)
        tq = hash_table.reshape(T // 128, 128, 2).transpose(0, 2, 1)
        tq = tq.reshape(2 * T)
        out_flat = _run(xt_flat, tq)
        return out_flat.reshape(N, F)


# speedup vs baseline: 196.6139x; 1.9359x over previous
import numpy as np
import jax
import jax.numpy as jnp
from jax import lax
from jax.experimental import pallas as pl
from jax.experimental.pallas import tpu as pltpu
from jax.experimental.pallas import tpu_sc as plsc

N = 1048576
T = 4194304
F = 2
_RES_MIN, _RES_MAX, _L, _LAYER = 16.0, 512.0, 16, 8
_GROWTH = np.exp((np.log(_RES_MAX) - np.log(_RES_MIN)) / (_L - 1))
RES = float(_GROWTH ** _LAYER * _RES_MIN)
SCALE = np.float32(RES - 1.0)
P1 = np.int32(2654435761 - (1 << 32))
P2 = np.int32(805459861)
MASK = np.int32(T - 1)

NC = 2
NS = 16
NW = NC * NS
NP = N // NW
B = 128
SB = 4096
CPS = SB // B
NSUPER = NP // SB


def _sc_body(xt, tq, out, x_buf, w_bufs, adr_bufs, val_bufs, out_buf, sem):
    wid = lax.axis_index("s") * NC + lax.axis_index("c")
    lane = lax.iota(jnp.int32, 16)
    one = jnp.float32(1.0)

    def hash_issue(k, p):
        cbase = k * B
        for g in range(B // 16):
            off = cbase + g * 16
            woff = g * 16
            s0 = x_buf[pl.ds(off, 16)] * SCALE
            s1 = x_buf[pl.ds(SB + off, 16)] * SCALE
            s2 = x_buf[pl.ds(2 * SB + off, 16)] * SCALE
            f0 = s0.astype(jnp.int32)
            f1 = s1.astype(jnp.int32)
            f2 = s2.astype(jnp.int32)
            f0 = jnp.where(f0.astype(jnp.float32) > s0, f0 - 1, f0)
            f1 = jnp.where(f1.astype(jnp.float32) > s1, f1 - 1, f1)
            f2 = jnp.where(f2.astype(jnp.float32) > s2, f2 - 1, f2)
            w_bufs[p][pl.ds(woff, 16)] = s0 - f0.astype(jnp.float32)
            w_bufs[p][pl.ds(B + woff, 16)] = s1 - f1.astype(jnp.float32)
            w_bufs[p][pl.ds(2 * B + woff, 16)] = s2 - f2.astype(jnp.float32)
            h0 = (f0, f0 + 1)
            h1f = f1 * P1
            h1 = (h1f, h1f + P1)
            h2f = f2 * P2
            h2 = (h2f, h2f + P2)
            for c in range(8):
                h = (h0[(c >> 2) & 1] ^ h1[(c >> 1) & 1] ^ h2[c & 1]) & MASK
                a0 = ((h >> 7) << 8) + (h & 127)
                adr_bufs[p][pl.ds(c * B + woff, 16)] = a0
                adr_bufs[p][pl.ds(8 * B + c * B + woff, 16)] = a0 + 128
        pltpu.async_copy(tq.at[adr_bufs[p]], val_bufs[p], sem[p])

    def wait_gather(p):
        pltpu.make_async_copy(tq.at[adr_bufs[p]], val_bufs[p], sem[p]).wait()

    def interp(k, p):
        cbase = k * B
        for j in range(B // 16):
            jo = j * 16
            w0 = w_bufs[p][pl.ds(jo, 16)]
            w1 = w_bufs[p][pl.ds(B + jo, 16)]
            w2 = w_bufs[p][pl.ds(2 * B + jo, 16)]
            u0, u1, u2 = one - w0, one - w1, one - w2
            res = []
            for f in range(2):
                v = [val_bufs[p][pl.ds(f * 8 * B + c * B + jo, 16)]
                     for c in range(8)]
                p00 = v[0] * u0 + v[4] * w0
                p01 = v[1] * u0 + v[5] * w0
                p10 = v[2] * u0 + v[6] * w0
                p11 = v[3] * u0 + v[7] * w0
                p0 = p00 * u1 + p10 * w1
                p1 = p01 * u1 + p11 * w1
                res.append(p0 * u2 + p1 * w2)
            out_buf[pl.ds(cbase * 2 + jo, 16)] = res[0]
            out_buf[pl.ds(cbase * 2 + 128 + jo, 16)] = res[1]

    def super_body(s, carry):
        sbase = wid * NP + s * SB
        for d in range(3):
            pltpu.sync_copy(xt.at[pl.ds(d * N + sbase, SB)],
                            x_buf.at[pl.ds(d * SB, SB)])

        hash_issue(jnp.int32(0), 0)

        def pair_body(m, carry2):
            k0 = m * 2
            hash_issue(k0 + 1, 1)
            wait_gather(0)
            interp(k0, 0)

            @pl.when(m < CPS // 2 - 1)
            def _():
                hash_issue(k0 + 2, 0)

            wait_gather(1)
            interp(k0 + 1, 1)
            return carry2

        lax.fori_loop(jnp.int32(0), jnp.int32(CPS // 2), pair_body,
                      jnp.int32(0))
        pltpu.sync_copy(out_buf, out.at[pl.ds(2 * sbase, 2 * SB)])
        return carry

    lax.fori_loop(jnp.int32(0), jnp.int32(NSUPER), super_body, jnp.int32(0))


@jax.jit
def _run(xt_flat, table8):
    fn = pl.kernel(
        _sc_body,
        out_type=jax.ShapeDtypeStruct((2 * N,), jnp.float32),
        mesh=plsc.VectorSubcoreMesh(
            core_axis_name="c", subcore_axis_name="s",
            num_cores=NC, num_subcores=NS,
        ),
        scratch_types=[
            pltpu.VMEM((3 * SB,), jnp.float32),
            [pltpu.VMEM((3 * B,), jnp.float32) for _ in range(2)],
            [pltpu.VMEM((16 * B,), jnp.int32) for _ in range(2)],
            [pltpu.VMEM((16 * B,), jnp.float32) for _ in range(2)],
            pltpu.VMEM((2 * SB,), jnp.float32),
            [pltpu.SemaphoreType.DMA for _ in range(2)],
        ],
        compiler_params=pltpu.CompilerParams(
            needs_layout_passes=False, use_tc_tiling_on_sc=False
        ),
    )
    return fn(xt_flat, table8)


def kernel(X, hash_table):
    with jax.enable_x64(False):
        xt_flat = X.T.reshape(3 * N).astype(jnp.float32)
        tq = hash_table.reshape(T // 128, 128, 2).transpose(0, 2, 1)
        tq = tq.reshape(2 * T)
        out_flat = _run(xt_flat, tq)
        out = out_flat.reshape(N // 128, 2, 128).transpose(0, 2, 1)
        return out.reshape(N, F)


# speedup vs baseline: 303.9454x; 1.5459x over previous
import numpy as np
import jax
import jax.numpy as jnp
from jax import lax
from jax.experimental import pallas as pl
from jax.experimental.pallas import tpu as pltpu
from jax.experimental.pallas import tpu_sc as plsc

N = 1048576
T = 4194304
F = 2
_RES_MIN, _RES_MAX, _L, _LAYER = 16.0, 512.0, 16, 8
_GROWTH = np.exp((np.log(_RES_MAX) - np.log(_RES_MIN)) / (_L - 1))
RES = float(_GROWTH ** _LAYER * _RES_MIN)
SCALE = np.float32(RES - 1.0)
P1 = np.int32(2654435761 - (1 << 32))
P2 = np.int32(805459861)
MASK = np.int32(T - 1)

NC = 2
NS = 16
NW = NC * NS
NP = N // NW
B = 128
SB = 4096
CPS = SB // B
NSUPER = NP // SB

RW = 2 * T // NW
RCH = 64
RIT = RW // (RCH * 256)


def _repack_body(tq, t8, in_buf, out_buf):
    wid = lax.axis_index("s") * NC + lax.axis_index("c")
    lane = lax.iota(jnp.int32, 16)
    pat = ((lane >> 2) & 1) * 128 + ((lane >> 3) << 2) + (lane & 3)
    w0 = wid * RW

    def it_body(it, carry):
        base = w0 + it * (RCH * 256)
        pltpu.sync_copy(tq.at[pl.ds(base, RCH * 256)], in_buf)

        def block_body(b, carry2):
            boff = b * 256
            for q in range(16):
                v = plsc.load_gather(in_buf, [pat + (boff + 8 * q)])
                out_buf[pl.ds(boff + q * 16, 16)] = v
            return carry2

        lax.fori_loop(jnp.int32(0), jnp.int32(RCH), block_body, jnp.int32(0))
        pltpu.sync_copy(out_buf, t8.at[pl.ds(base, RCH * 256)])
        return carry

    lax.fori_loop(jnp.int32(0), jnp.int32(RIT), it_body, jnp.int32(0))


def _sc_body(xt, t8, out, x_buf, w_bufs, adr_bufs, cb_bufs, val_bufs,
             out_buf, sem):
    wid = lax.axis_index("s") * NC + lax.axis_index("c")
    lane = lax.iota(jnp.int32, 16)
    one = jnp.float32(1.0)

    def hash_issue(k, p):
        cbase = k * B
        for g in range(B // 16):
            off = cbase + g * 16
            woff = g * 16
            s0 = x_buf[pl.ds(off, 16)] * SCALE
            s1 = x_buf[pl.ds(SB + off, 16)] * SCALE
            s2 = x_buf[pl.ds(2 * SB + off, 16)] * SCALE
            f0 = s0.astype(jnp.int32)
            f1 = s1.astype(jnp.int32)
            f2 = s2.astype(jnp.int32)
            f0 = jnp.where(f0.astype(jnp.float32) > s0, f0 - 1, f0)
            f1 = jnp.where(f1.astype(jnp.float32) > s1, f1 - 1, f1)
            f2 = jnp.where(f2.astype(jnp.float32) > s2, f2 - 1, f2)
            w_bufs[p][pl.ds(woff, 16)] = s0 - f0.astype(jnp.float32)
            w_bufs[p][pl.ds(B + woff, 16)] = s1 - f1.astype(jnp.float32)
            w_bufs[p][pl.ds(2 * B + woff, 16)] = s2 - f2.astype(jnp.float32)
            h0 = (f0, f0 + 1)
            h1f = f1 * P1
            h1 = (h1f, h1f + P1)
            h2f = f2 * P2
            h2 = (h2f, h2f + P2)
            for c in range(8):
                h = (h0[(c >> 2) & 1] ^ h1[(c >> 1) & 1] ^ h2[c & 1]) & MASK
                adr_bufs[p][pl.ds(c * B + woff, 16)] = h >> 2
                cb_bufs[p][pl.ds(c * B + woff, 16)] = h & 3
        pltpu.async_copy(t8.at[adr_bufs[p]], val_bufs[p], sem[p])

    def wait_gather(p):
        pltpu.make_async_copy(t8.at[adr_bufs[p]], val_bufs[p], sem[p]).wait()

    def interp(k, p):
        cbase = k * B
        for j in range(B // 16):
            jo = j * 16
            w0 = w_bufs[p][pl.ds(jo, 16)]
            w1 = w_bufs[p][pl.ds(B + jo, 16)]
            w2 = w_bufs[p][pl.ds(2 * B + jo, 16)]
            u0, u1, u2 = one - w0, one - w1, one - w2
            res = []
            for f in range(2):
                v = []
                for c in range(8):
                    rows = lane + (c * B + jo)
                    cols = cb_bufs[p][pl.ds(c * B + jo, 16)] + f * 4
                    v.append(plsc.load_gather(val_bufs[p], [rows, cols]))
                p00 = v[0] * u0 + v[4] * w0
                p01 = v[1] * u0 + v[5] * w0
                p10 = v[2] * u0 + v[6] * w0
                p11 = v[3] * u0 + v[7] * w0
                p0 = p00 * u1 + p10 * w1
                p1 = p01 * u1 + p11 * w1
                res.append(p0 * u2 + p1 * w2)
            out_buf[pl.ds(cbase * 2 + jo, 16)] = res[0]
            out_buf[pl.ds(cbase * 2 + 128 + jo, 16)] = res[1]

    def super_body(s, carry):
        sbase = wid * NP + s * SB
        for d in range(3):
            pltpu.sync_copy(xt.at[pl.ds(d * N + sbase, SB)],
                            x_buf.at[pl.ds(d * SB, SB)])

        hash_issue(jnp.int32(0), 0)

        def pair_body(m, carry2):
            k0 = m * 2
            hash_issue(k0 + 1, 1)
            wait_gather(0)
            interp(k0, 0)

            @pl.when(m < CPS // 2 - 1)
            def _():
                hash_issue(k0 + 2, 0)

            wait_gather(1)
            interp(k0 + 1, 1)
            return carry2

        lax.fori_loop(jnp.int32(0), jnp.int32(CPS // 2), pair_body,
                      jnp.int32(0))
        pltpu.sync_copy(out_buf, out.at[pl.ds(2 * sbase, 2 * SB)])
        return carry

    lax.fori_loop(jnp.int32(0), jnp.int32(NSUPER), super_body, jnp.int32(0))


_SC_PARAMS = pltpu.CompilerParams(
    needs_layout_passes=False, use_tc_tiling_on_sc=False
)
_MESH = dict(core_axis_name="c", subcore_axis_name="s",
             num_cores=NC, num_subcores=NS)


@jax.jit
def _run(xt_flat, tq):
    repack = pl.kernel(
        _repack_body,
        out_type=jax.ShapeDtypeStruct((2 * T,), jnp.float32),
        mesh=plsc.VectorSubcoreMesh(**_MESH),
        scratch_types=[
            pltpu.VMEM((RCH * 256,), jnp.float32),
            pltpu.VMEM((RCH * 256,), jnp.float32),
        ],
        compiler_params=_SC_PARAMS,
    )
    t8 = repack(tq).reshape(T // 4, 8)

    fn = pl.kernel(
        _sc_body,
        out_type=jax.ShapeDtypeStruct((2 * N,), jnp.float32),
        mesh=plsc.VectorSubcoreMesh(**_MESH),
        scratch_types=[
            pltpu.VMEM((3 * SB,), jnp.float32),
            [pltpu.VMEM((3 * B,), jnp.float32) for _ in range(2)],
            [pltpu.VMEM((8 * B,), jnp.int32) for _ in range(2)],
            [pltpu.VMEM((8 * B,), jnp.int32) for _ in range(2)],
            [pltpu.VMEM((8 * B, 8), jnp.float32) for _ in range(2)],
            pltpu.VMEM((2 * SB,), jnp.float32),
            [pltpu.SemaphoreType.DMA for _ in range(2)],
        ],
        compiler_params=_SC_PARAMS,
    )
    return fn(xt_flat, t8)


def kernel(X, hash_table):
    with jax.enable_x64(False):
        xt_flat = X.T.reshape(3 * N).astype(jnp.float32)
        tq = hash_table.reshape(T // 128, 128, 2).transpose(0, 2, 1)
        tq = tq.reshape(2 * T)
        out_flat = _run(xt_flat, tq)
        out = out_flat.reshape(N // 128, 2, 128).transpose(0, 2, 1)
        return out.reshape(N, F)


# speedup vs baseline: 319.6805x; 1.0518x over previous
import numpy as np
import jax
import jax.numpy as jnp
from jax import lax
from jax.experimental import pallas as pl
from jax.experimental.pallas import tpu as pltpu
from jax.experimental.pallas import tpu_sc as plsc

N = 1048576
T = 4194304
F = 2
_RES_MIN, _RES_MAX, _L, _LAYER = 16.0, 512.0, 16, 8
_GROWTH = np.exp((np.log(_RES_MAX) - np.log(_RES_MIN)) / (_L - 1))
RES = float(_GROWTH ** _LAYER * _RES_MIN)
SCALE = np.float32(RES - 1.0)
P1 = np.int32(2654435761 - (1 << 32))
P2 = np.int32(805459861)
MASK = np.int32(T - 1)

NC = 2
NS = 16
NW = NC * NS
NP = N // NW
B = 128
SB = 4096
CPS = SB // B
NSUPER = NP // SB

RW = 2 * T // NW
RCH = 64
RIT = RW // (RCH * 256)


def _repack_body(tq, t8, in_bufs, out_bufs, isem, osem):
    wid = lax.axis_index("s") * NC + lax.axis_index("c")
    lane = lax.iota(jnp.int32, 16)
    pat = ((lane >> 2) & 1) * 128 + ((lane >> 3) << 2) + (lane & 3)
    w0 = wid * RW
    CW = RCH * 256

    def issue_in(it, p):
        pltpu.async_copy(tq.at[pl.ds(w0 + it * CW, CW)], in_bufs[p], isem[p])

    def wait_in(it, p):
        pltpu.make_async_copy(tq.at[pl.ds(w0 + it * CW, CW)], in_bufs[p],
                              isem[p]).wait()

    def issue_out(it, p):
        pltpu.async_copy(out_bufs[p], t8.at[pl.ds(w0 + it * CW, CW)], osem[p])

    def wait_out(it, p):
        pltpu.make_async_copy(out_bufs[p], t8.at[pl.ds(w0 + it * CW, CW)],
                              osem[p]).wait()

    def compute(p):
        def block_body(b, carry2):
            boff = b * 256
            for q in range(16):
                v = plsc.load_gather(in_bufs[p], [pat + (boff + 8 * q)])
                out_bufs[p][pl.ds(boff + q * 16, 16)] = v
            return carry2

        lax.fori_loop(jnp.int32(0), jnp.int32(RCH), block_body, jnp.int32(0))

    issue_in(jnp.int32(0), 0)

    def pair_body(m, carry):
        it0 = m * 2
        for j, p in ((0, 0), (1, 1)):
            it = it0 + j
            wait_in(it, p)

            @pl.when(it + 1 < RIT)
            def _():
                issue_in(it + 1, 1 - p)

            @pl.when(m > 0)
            def _():
                wait_out(it - 2, p)

            compute(p)
            issue_out(it, p)
        return carry

    lax.fori_loop(jnp.int32(0), jnp.int32(RIT // 2), pair_body, jnp.int32(0))
    wait_out(jnp.int32(RIT - 2), 0)
    wait_out(jnp.int32(RIT - 1), 1)


def _sc_body(xt, t8, out, x_buf, w_bufs, adr_bufs, cb_bufs, val_bufs,
             out_buf, sem):
    wid = lax.axis_index("s") * NC + lax.axis_index("c")
    lane = lax.iota(jnp.int32, 16)
    one = jnp.float32(1.0)

    def hash_issue(k, p):
        cbase = k * B
        for g in range(B // 16):
            off = cbase + g * 16
            woff = g * 16
            s0 = x_buf[pl.ds(off, 16)] * SCALE
            s1 = x_buf[pl.ds(SB + off, 16)] * SCALE
            s2 = x_buf[pl.ds(2 * SB + off, 16)] * SCALE
            f0 = s0.astype(jnp.int32)
            f1 = s1.astype(jnp.int32)
            f2 = s2.astype(jnp.int32)
            f0 = jnp.where(f0.astype(jnp.float32) > s0, f0 - 1, f0)
            f1 = jnp.where(f1.astype(jnp.float32) > s1, f1 - 1, f1)
            f2 = jnp.where(f2.astype(jnp.float32) > s2, f2 - 1, f2)
            w_bufs[p][pl.ds(woff, 16)] = s0 - f0.astype(jnp.float32)
            w_bufs[p][pl.ds(B + woff, 16)] = s1 - f1.astype(jnp.float32)
            w_bufs[p][pl.ds(2 * B + woff, 16)] = s2 - f2.astype(jnp.float32)
            h0 = (f0, f0 + 1)
            h1f = f1 * P1
            h1 = (h1f, h1f + P1)
            h2f = f2 * P2
            h2 = (h2f, h2f + P2)
            for c in range(8):
                h = (h0[(c >> 2) & 1] ^ h1[(c >> 1) & 1] ^ h2[c & 1]) & MASK
                adr_bufs[p][pl.ds(c * B + woff, 16)] = h >> 2
                cb_bufs[p][pl.ds(c * B + woff, 16)] = h & 3
        pltpu.async_copy(t8.at[adr_bufs[p]], val_bufs[p], sem[p])

    def wait_gather(p):
        pltpu.make_async_copy(t8.at[adr_bufs[p]], val_bufs[p], sem[p]).wait()

    def interp(k, p):
        cbase = k * B
        for j in range(B // 16):
            jo = j * 16
            w0 = w_bufs[p][pl.ds(jo, 16)]
            w1 = w_bufs[p][pl.ds(B + jo, 16)]
            w2 = w_bufs[p][pl.ds(2 * B + jo, 16)]
            u0, u1, u2 = one - w0, one - w1, one - w2
            res = []
            for f in range(2):
                v = []
                for c in range(8):
                    rows = lane + (c * B + jo)
                    cols = cb_bufs[p][pl.ds(c * B + jo, 16)] + f * 4
                    v.append(plsc.load_gather(val_bufs[p], [rows, cols]))
                p00 = v[0] * u0 + v[4] * w0
                p01 = v[1] * u0 + v[5] * w0
                p10 = v[2] * u0 + v[6] * w0
                p11 = v[3] * u0 + v[7] * w0
                p0 = p00 * u1 + p10 * w1
                p1 = p01 * u1 + p11 * w1
                res.append(p0 * u2 + p1 * w2)
            out_buf[pl.ds(cbase * 2 + jo, 16)] = res[0]
            out_buf[pl.ds(cbase * 2 + 128 + jo, 16)] = res[1]

    def super_body(s, carry):
        sbase = wid * NP + s * SB
        for d in range(3):
            pltpu.sync_copy(xt.at[pl.ds(d * N + sbase, SB)],
                            x_buf.at[pl.ds(d * SB, SB)])

        hash_issue(jnp.int32(0), 0)
        hash_issue(jnp.int32(1), 1)
        hash_issue(jnp.int32(2), 2)

        def quad_body(m, carry2):
            k0 = m * 4
            for j in range(4):
                k = k0 + j

                @pl.when(k + 3 < CPS)
                def _():
                    hash_issue(k + 3, (j + 3) % 4)

                wait_gather(j)
                interp(k, j)
            return carry2

        lax.fori_loop(jnp.int32(0), jnp.int32(CPS // 4), quad_body,
                      jnp.int32(0))
        pltpu.sync_copy(out_buf, out.at[pl.ds(2 * sbase, 2 * SB)])
        return carry

    lax.fori_loop(jnp.int32(0), jnp.int32(NSUPER), super_body, jnp.int32(0))


_SC_PARAMS = pltpu.CompilerParams(
    needs_layout_passes=False, use_tc_tiling_on_sc=False
)
_MESH = dict(core_axis_name="c", subcore_axis_name="s",
             num_cores=NC, num_subcores=NS)


@jax.jit
def _run(xt_flat, tq):
    repack = pl.kernel(
        _repack_body,
        out_type=jax.ShapeDtypeStruct((2 * T,), jnp.float32),
        mesh=plsc.VectorSubcoreMesh(**_MESH),
        scratch_types=[
            [pltpu.VMEM((RCH * 256,), jnp.float32) for _ in range(2)],
            [pltpu.VMEM((RCH * 256,), jnp.float32) for _ in range(2)],
            [pltpu.SemaphoreType.DMA for _ in range(2)],
            [pltpu.SemaphoreType.DMA for _ in range(2)],
        ],
        compiler_params=_SC_PARAMS,
    )
    t8 = repack(tq).reshape(T // 4, 8)

    fn = pl.kernel(
        _sc_body,
        out_type=jax.ShapeDtypeStruct((2 * N,), jnp.float32),
        mesh=plsc.VectorSubcoreMesh(**_MESH),
        scratch_types=[
            pltpu.VMEM((3 * SB,), jnp.float32),
            [pltpu.VMEM((3 * B,), jnp.float32) for _ in range(4)],
            [pltpu.VMEM((8 * B,), jnp.int32) for _ in range(4)],
            [pltpu.VMEM((8 * B,), jnp.int32) for _ in range(4)],
            [pltpu.VMEM((8 * B, 8), jnp.float32) for _ in range(4)],
            pltpu.VMEM((2 * SB,), jnp.float32),
            [pltpu.SemaphoreType.DMA for _ in range(4)],
        ],
        compiler_params=_SC_PARAMS,
    )
    return fn(xt_flat, t8)


def kernel(X, hash_table):
    with jax.enable_x64(False):
        xt_flat = X.T.reshape(3 * N).astype(jnp.float32)
        tq = hash_table.reshape(T // 128, 128, 2).transpose(0, 2, 1)
        tq = tq.reshape(2 * T)
        out_flat = _run(xt_flat, tq)
        out = out_flat.reshape(N // 128, 2, 128).transpose(0, 2, 1)
        return out.reshape(N, F)
